# baseline (speedup 1.0000x reference)
import jax
import jax.numpy as jnp
from jax.experimental import pallas as pl
from jax.experimental.pallas import tpu as pltpu

ORDER = 16
SUBSAMPLING_FACTOR = 2
NUM_FEATURES = 64
SUB_ORDER = ORDER // SUBSAMPLING_FACTOR


def _dma_kernel(x_hbm, out_hbm, sem):
    B = x_hbm.shape[0]
    cp = pltpu.make_async_copy(
        x_hbm.at[:, :, pl.ds(0, 1)],
        out_hbm,
        sem,
    )
    cp.start()
    cp.wait()


def kernel(x):
    B, C, H, W = x.shape
    xs = x.reshape(B, SUB_ORDER, SUBSAMPLING_FACTOR, NUM_FEATURES, H, W)
    out = pl.pallas_call(
        _dma_kernel,
        in_specs=[pl.BlockSpec(memory_space=pltpu.MemorySpace.HBM)],
        out_specs=pl.BlockSpec(memory_space=pltpu.MemorySpace.HBM),
        out_shape=jax.ShapeDtypeStruct((B, SUB_ORDER, 1, NUM_FEATURES, H, W), x.dtype),
        scratch_shapes=[pltpu.SemaphoreType.DMA],
    )(xs)
    return out.reshape(B, SUB_ORDER * NUM_FEATURES, H, W)


# speedup vs baseline: 16.0970x; 16.0970x over previous
import jax
import jax.numpy as jnp
from jax.experimental import pallas as pl
from jax.experimental.pallas import tpu as pltpu

ORDER = 16
SUBSAMPLING_FACTOR = 2
NUM_FEATURES = 64
SUB_ORDER = ORDER // SUBSAMPLING_FACTOR

NBUF = 2
DEPTH = 1


def _dma_kernel(x_hbm, out_hbm, *bufs_and_sems):
    bufs = bufs_and_sems[:NBUF]
    sin = bufs_and_sems[NBUF : 2 * NBUF]
    sout = bufs_and_sems[2 * NBUF :]
    B = x_hbm.shape[0]

    def mk(t):
        r = t % NBUF
        cin = pltpu.make_async_copy(
            x_hbm.at[pl.ds(t, 1), :, pl.ds(0, 1)], bufs[r], sin[r]
        )
        cout = pltpu.make_async_copy(bufs[r], out_hbm.at[pl.ds(t, 1)], sout[r])
        return cin, cout

    copies = [mk(t) for t in range(B)]
    out_waited = [False] * B
    for t in range(min(DEPTH, B)):
        copies[t][0].start()
    for t in range(B):
        copies[t][0].wait()
        copies[t][1].start()
        u = t + DEPTH
        if u < B:
            if u >= NBUF:
                copies[u - NBUF][1].wait()
                out_waited[u - NBUF] = True
            copies[u][0].start()
    for t in range(B):
        if not out_waited[t]:
            copies[t][1].wait()


def kernel(x):
    B, C, H, W = x.shape
    xs = x.reshape(B, SUB_ORDER, SUBSAMPLING_FACTOR, NUM_FEATURES, H, W)
    out = pl.pallas_call(
        _dma_kernel,
        in_specs=[pl.BlockSpec(memory_space=pltpu.MemorySpace.HBM)],
        out_specs=pl.BlockSpec(memory_space=pltpu.MemorySpace.HBM),
        out_shape=jax.ShapeDtypeStruct((B, SUB_ORDER, 1, NUM_FEATURES, H, W), x.dtype),
        scratch_shapes=[
            pltpu.VMEM((1, SUB_ORDER, 1, NUM_FEATURES, H, W), jnp.float32)
            for _ in range(NBUF)
        ]
        + [pltpu.SemaphoreType.DMA] * (2 * NBUF),
    )(xs)
    return out.reshape(B, SUB_ORDER * NUM_FEATURES, H, W)


# speedup vs baseline: 16.4884x; 1.0243x over previous
import jax
import jax.numpy as jnp
from jax.experimental import pallas as pl
from jax.experimental.pallas import tpu as pltpu

ORDER = 16
SUBSAMPLING_FACTOR = 2
NUM_FEATURES = 64
SUB_ORDER = ORDER // SUBSAMPLING_FACTOR

CHB = 4
NBUF = 6
DEPTH = 3


def _dma_kernel(x_hbm, out_hbm, *bufs_and_sems):
    bufs = bufs_and_sems[:NBUF]
    sin = bufs_and_sems[NBUF : 2 * NBUF]
    sout = bufs_and_sems[2 * NBUF :]
    B = x_hbm.shape[0]
    gpb = SUB_ORDER // CHB
    n_chunks = B * gpb

    def mk(t):
        b, g = divmod(t, gpb)
        r = t % NBUF
        cin = pltpu.make_async_copy(
            x_hbm.at[pl.ds(b, 1), pl.ds(g * CHB, CHB), pl.ds(0, 1)],
            bufs[r],
            sin[r],
        )
        cout = pltpu.make_async_copy(
            bufs[r], out_hbm.at[pl.ds(b, 1), pl.ds(g * CHB, CHB)], sout[r]
        )
        return cin, cout

    copies = [mk(t) for t in range(n_chunks)]
    out_waited = [False] * n_chunks
    for t in range(min(DEPTH, n_chunks)):
        copies[t][0].start()
    for t in range(n_chunks):
        copies[t][0].wait()
        copies[t][1].start()
        u = t + DEPTH
        if u < n_chunks:
            if u >= NBUF:
                copies[u - NBUF][1].wait()
                out_waited[u - NBUF] = True
            copies[u][0].start()
    for t in range(n_chunks):
        if not out_waited[t]:
            copies[t][1].wait()


def kernel(x):
    B, C, H, W = x.shape
    xs = x.reshape(B, SUB_ORDER, SUBSAMPLING_FACTOR, NUM_FEATURES, H, W)
    out = pl.pallas_call(
        _dma_kernel,
        in_specs=[pl.BlockSpec(memory_space=pltpu.MemorySpace.HBM)],
        out_specs=pl.BlockSpec(memory_space=pltpu.MemorySpace.HBM),
        out_shape=jax.ShapeDtypeStruct((B, SUB_ORDER, 1, NUM_FEATURES, H, W), x.dtype),
        scratch_shapes=[
            pltpu.VMEM((1, CHB, 1, NUM_FEATURES, H, W), jnp.float32)
            for _ in range(NBUF)
        ]
        + [pltpu.SemaphoreType.DMA] * (2 * NBUF),
    )(xs)
    return out.reshape(B, SUB_ORDER * NUM_FEATURES, H, W)


# speedup vs baseline: 16.4980x; 1.0006x over previous
import jax
import jax.numpy as jnp
from jax.experimental import pallas as pl
from jax.experimental.pallas import tpu as pltpu

ORDER = 16
SUBSAMPLING_FACTOR = 2
NUM_FEATURES = 64
SUB_ORDER = ORDER // SUBSAMPLING_FACTOR

CHB = 2
NBUF = 12
DEPTH = 6


def _dma_kernel(x_hbm, out_hbm, *bufs_and_sems):
    bufs = bufs_and_sems[:NBUF]
    sin = bufs_and_sems[NBUF : 2 * NBUF]
    sout = bufs_and_sems[2 * NBUF :]
    B = x_hbm.shape[0]
    gpb = SUB_ORDER // CHB
    n_chunks = B * gpb

    def mk(t):
        b, g = divmod(t, gpb)
        r = t % NBUF
        cin = pltpu.make_async_copy(
            x_hbm.at[pl.ds(b, 1), pl.ds(g * CHB, CHB), pl.ds(0, 1)],
            bufs[r],
            sin[r],
        )
        cout = pltpu.make_async_copy(
            bufs[r], out_hbm.at[pl.ds(b, 1), pl.ds(g * CHB, CHB)], sout[r]
        )
        return cin, cout

    copies = [mk(t) for t in range(n_chunks)]
    out_waited = [False] * n_chunks
    for t in range(min(DEPTH, n_chunks)):
        copies[t][0].start()
    for t in range(n_chunks):
        copies[t][0].wait()
        copies[t][1].start()
        u = t + DEPTH
        if u < n_chunks:
            if u >= NBUF:
                copies[u - NBUF][1].wait()
                out_waited[u - NBUF] = True
            copies[u][0].start()
    for t in range(n_chunks):
        if not out_waited[t]:
            copies[t][1].wait()


def kernel(x):
    B, C, H, W = x.shape
    xs = x.reshape(B, SUB_ORDER, SUBSAMPLING_FACTOR, NUM_FEATURES, H, W)
    out = pl.pallas_call(
        _dma_kernel,
        in_specs=[pl.BlockSpec(memory_space=pltpu.MemorySpace.HBM)],
        out_specs=pl.BlockSpec(memory_space=pltpu.MemorySpace.HBM),
        out_shape=jax.ShapeDtypeStruct((B, SUB_ORDER, 1, NUM_FEATURES, H, W), x.dtype),
        scratch_shapes=[
            pltpu.VMEM((1, CHB, 1, NUM_FEATURES, H, W), jnp.float32)
            for _ in range(NBUF)
        ]
        + [pltpu.SemaphoreType.DMA] * (2 * NBUF),
    )(xs)
    return out.reshape(B, SUB_ORDER * NUM_FEATURES, H, W)
